# Initial kernel scaffold; baseline (speedup 1.0000x reference)
import jax
import jax.numpy as jnp
from jax.experimental import pallas as pl


def kernel(op_idx, a, b, c, op_embed, Wp, bp, tile_keys, W1, b1, W2, b2, Wr, br, Wf, bf):
    raise NotImplementedError("write your pallas kernel here")


# speedup vs baseline: 1.1568x; 1.1568x over previous
import functools

import jax
import jax.numpy as jnp
from jax.experimental import pallas as pl
from jax.experimental.pallas import tpu as pltpu

B = 4096
D_MODEL = 512
NUM_TILES = 16
TOP_K = 4
D_FF = 1024
BLK = 512
NBLK = B // BLK


def _routing_body(op_ref, a_ref, b_ref, c_ref, emb_ref, wp_ref, bp_ref,
                  keys_ref, x_ref, dgt_ref, idxt_ref, aux_ref,
                  psum_acc, cnt_acc):
    i = pl.program_id(0)

    op_col = op_ref[0]
    a_col = a_ref[0]
    b_col = b_ref[0]
    c_col = c_ref[0]

    iota8 = jax.lax.broadcasted_iota(jnp.int32, (BLK, 8), 1)
    onehot_op = (op_col == iota8).astype(jnp.float32)
    op_emb = jnp.dot(onehot_op, emb_ref[...],
                     preferred_element_type=jnp.float32)
    a_bits = ((a_col >> iota8) & 1).astype(jnp.float32)
    b_bits = ((b_col >> iota8) & 1).astype(jnp.float32)
    zeros7 = jnp.zeros((BLK, 7), jnp.float32)
    feats = jnp.concatenate(
        [op_emb, a_bits, b_bits, c_col.astype(jnp.float32), zeros7], axis=1)

    x = jnp.dot(feats, wp_ref[...],
                preferred_element_type=jnp.float32) + bp_ref[...]
    x_ref[...] = x

    logits_t = jax.lax.dot_general(
        keys_ref[...], x, (((1,), (1,)), ((), ())),
        preferred_element_type=jnp.float32)

    iota_e = jax.lax.broadcasted_iota(jnp.int32, (NUM_TILES, BLK), 0)

    cur = logits_t
    vals_rows = []
    idx_rows = []
    for _ in range(TOP_K):
        m = jnp.max(cur, axis=0, keepdims=True)
        is_max = cur == m
        am = jnp.min(jnp.where(is_max, iota_e, NUM_TILES),
                     axis=0, keepdims=True)
        vals_rows.append(m)
        idx_rows.append(am)
        cur = jnp.where(iota_e == am, -jnp.inf, cur)

    vcat = jnp.concatenate(vals_rows, axis=0)
    ecat = jnp.exp(vcat - vals_rows[0])
    gates_t = ecat / jnp.sum(ecat, axis=0, keepdims=True)

    dgt = jnp.zeros((NUM_TILES, BLK), jnp.float32)
    for k in range(TOP_K):
        dgt = dgt + jnp.where(iota_e == idx_rows[k],
                              gates_t[k:k + 1, :], 0.0)
    dgt_ref[...] = dgt
    idxt_ref[...] = jnp.concatenate(idx_rows, axis=0)

    pe = jnp.exp(logits_t - jnp.max(logits_t, axis=0, keepdims=True))
    probs_t = pe / jnp.sum(pe, axis=0, keepdims=True)
    psum = jnp.sum(probs_t, axis=1, keepdims=True)
    cnt = jnp.sum((dgt > 0.0).astype(jnp.float32), axis=1, keepdims=True)

    @pl.when(i == 0)
    def _init():
        psum_acc[...] = jnp.zeros_like(psum_acc)
        cnt_acc[...] = jnp.zeros_like(cnt_acc)

    psum_acc[...] += jnp.broadcast_to(psum, psum_acc.shape)
    cnt_acc[...] += jnp.broadcast_to(cnt, cnt_acc.shape)

    @pl.when(i == NBLK - 1)
    def _fin():
        prod = psum_acc[:, 0:1] * cnt_acc[:, 0:1]
        s = jnp.sum(prod, keepdims=True)
        aux_ref[...] = s * (NUM_TILES / (B * float(B)))


def _ffn_body(x_ref, w1_ref, b1_ref, w2_ref, b2_ref, dgt_ref,
              wr_ref, br_ref, wf_ref, bf_ref,
              res_ref, flg_ref, out_acc):
    e = pl.program_id(1)

    @pl.when(e == 0)
    def _init():
        out_acc[...] = jnp.zeros_like(out_acc)

    x = x_ref[...]
    h = jnp.dot(x, w1_ref[0], preferred_element_type=jnp.float32)
    h = jax.nn.gelu(h + b1_ref[0])
    y = jnp.dot(h, w2_ref[0], preferred_element_type=jnp.float32)
    y = y + b2_ref[0]

    iota_e = jax.lax.broadcasted_iota(jnp.int32, (NUM_TILES, 1), 0)
    onehot_e = (iota_e == e).astype(jnp.float32)
    g_col = jax.lax.dot_general(
        dgt_ref[...], onehot_e, (((0,), (0,)), ((), ())),
        preferred_element_type=jnp.float32)

    out_acc[...] += g_col * y

    @pl.when(e == NUM_TILES - 1)
    def _heads():
        out = out_acc[...]
        res_ref[...] = jax.nn.sigmoid(
            jnp.dot(out, wr_ref[...], preferred_element_type=jnp.float32)
            + br_ref[...])
        flg_ref[...] = jax.nn.sigmoid(
            jnp.dot(out, wf_ref[...], preferred_element_type=jnp.float32)
            + bf_ref[...])


def kernel(op_idx, a, b, c, op_embed, Wp, bp, tile_keys, W1, b1, W2, b2,
           Wr, br, Wf, bf):
    f32 = jnp.float32
    op_r = op_idx.astype(jnp.int32).reshape(NBLK, BLK, 1)
    a_r = a.astype(jnp.int32).reshape(NBLK, BLK, 1)
    b_r = b.astype(jnp.int32).reshape(NBLK, BLK, 1)
    c_r = c.astype(jnp.int32).reshape(NBLK, BLK, 1)
    wp_pad = jnp.zeros((40, D_MODEL), f32).at[:33].set(Wp)

    col_spec = pl.BlockSpec((1, BLK, 1), lambda i: (i, 0, 0))
    x_out, dgt, idx_t, aux = pl.pallas_call(
        _routing_body,
        grid=(NBLK,),
        in_specs=[
            col_spec, col_spec, col_spec, col_spec,
            pl.BlockSpec((8, 16), lambda i: (0, 0)),
            pl.BlockSpec((40, D_MODEL), lambda i: (0, 0)),
            pl.BlockSpec((1, D_MODEL), lambda i: (0, 0)),
            pl.BlockSpec((NUM_TILES, D_MODEL), lambda i: (0, 0)),
        ],
        out_specs=[
            pl.BlockSpec((BLK, D_MODEL), lambda i: (i, 0)),
            pl.BlockSpec((NUM_TILES, BLK), lambda i: (0, i)),
            pl.BlockSpec((TOP_K, BLK), lambda i: (0, i)),
            pl.BlockSpec((1, 1), lambda i: (0, 0)),
        ],
        out_shape=[
            jax.ShapeDtypeStruct((B, D_MODEL), f32),
            jax.ShapeDtypeStruct((NUM_TILES, B), f32),
            jax.ShapeDtypeStruct((TOP_K, B), jnp.int32),
            jax.ShapeDtypeStruct((1, 1), f32),
        ],
        scratch_shapes=[
            pltpu.VMEM((NUM_TILES, 128), f32),
            pltpu.VMEM((NUM_TILES, 128), f32),
        ],
    )(op_r, a_r, b_r, c_r, op_embed, wp_pad, bp.reshape(1, D_MODEL),
      tile_keys)

    result, flags = pl.pallas_call(
        _ffn_body,
        grid=(NBLK, NUM_TILES),
        in_specs=[
            pl.BlockSpec((BLK, D_MODEL), lambda i, e: (i, 0)),
            pl.BlockSpec((1, D_MODEL, D_FF), lambda i, e: (e, 0, 0)),
            pl.BlockSpec((1, 1, D_FF), lambda i, e: (e, 0, 0)),
            pl.BlockSpec((1, D_FF, D_MODEL), lambda i, e: (e, 0, 0)),
            pl.BlockSpec((1, 1, D_MODEL), lambda i, e: (e, 0, 0)),
            pl.BlockSpec((NUM_TILES, BLK), lambda i, e: (0, i)),
            pl.BlockSpec((D_MODEL, 8), lambda i, e: (0, 0)),
            pl.BlockSpec((1, 8), lambda i, e: (0, 0)),
            pl.BlockSpec((D_MODEL, 2), lambda i, e: (0, 0)),
            pl.BlockSpec((1, 2), lambda i, e: (0, 0)),
        ],
        out_specs=[
            pl.BlockSpec((BLK, 8), lambda i, e: (i, 0)),
            pl.BlockSpec((BLK, 2), lambda i, e: (i, 0)),
        ],
        out_shape=[
            jax.ShapeDtypeStruct((B, 8), f32),
            jax.ShapeDtypeStruct((B, 2), f32),
        ],
        scratch_shapes=[pltpu.VMEM((BLK, D_MODEL), f32)],
    )(x_out, W1, b1.reshape(NUM_TILES, 1, D_FF), W2,
      b2.reshape(NUM_TILES, 1, D_MODEL), dgt, Wr, br.reshape(1, 8),
      Wf, bf.reshape(1, 2))

    idx = idx_t.T
    return result, flags, idx, aux.reshape(())
